# baseline (speedup 1.0000x reference)
import functools

import jax
import jax.numpy as jnp
from jax import lax
from jax.experimental import pallas as pl
from jax.experimental.pallas import tpu as pltpu
from jax.experimental.pallas import tpu_sc as plsc

S = 2048
E = 16
C = 256
D = 1024
F = 4096
NSLOT = E * C

NW = 32
SLOTS_PER_W = NSLOT // NW
TOK_PER_W = S // NW

_HI = jax.lax.Precision.HIGHEST


def _gating_body(x_ref, wg_ref, slot1_ref, slot2_ref, tfs_ref, gfs_ref):
    logits = jnp.dot(x_ref[...], wg_ref[...],
                     preferred_element_type=jnp.float32)
    gates = jax.nn.softmax(logits, axis=-1)

    eio = lax.broadcasted_iota(jnp.int32, (S, E), 1)
    m1 = jnp.max(gates, axis=1, keepdims=True)
    idx1 = jnp.min(jnp.where(gates == m1, eio, E), axis=1, keepdims=True)
    mask1 = (eio == idx1).astype(jnp.float32)
    logits2 = jnp.where(mask1 > 0, -jnp.inf, logits)
    m2 = jnp.max(logits2, axis=1, keepdims=True)
    idx2 = jnp.min(jnp.where(logits2 == m2, eio, E), axis=1, keepdims=True)
    mask2 = (eio == idx2).astype(jnp.float32)

    count1 = jnp.sum(mask1, axis=0, keepdims=True)

    NB = 8
    BS = S // NB
    tril = (lax.broadcasted_iota(jnp.int32, (BS, BS), 0)
            >= lax.broadcasted_iota(jnp.int32, (BS, BS), 1)
            ).astype(jnp.float32)

    carry1 = jnp.zeros((1, E), jnp.float32)
    carry2 = jnp.zeros((1, E), jnp.float32)
    tfs_acc = jnp.zeros((E, C), jnp.float32)
    gfs_acc = jnp.zeros((E, C), jnp.float32)
    used = jnp.zeros((1, E), jnp.float32)
    raw1_l, raw2_l, k1_l, k2_l = [], [], [], []

    cio = lax.broadcasted_iota(jnp.int32, (BS, C), 1)

    for b in range(NB):
        sl = slice(b * BS, (b + 1) * BS)
        m1b, m2b, gb = mask1[sl], mask2[sl], gates[sl]
        e1b, e2b = idx1[sl], idx2[sl]
        m12 = jnp.concatenate([m1b, m2b], axis=1)
        cum12 = (jnp.dot(tril, m12, precision=_HI)
                 + jnp.concatenate([carry1, carry2], axis=1))
        cum1 = cum12[:, :E]
        cum2 = cum12[:, E:]
        carry1 = carry1 + jnp.sum(m1b, axis=0, keepdims=True)
        carry2 = carry2 + jnp.sum(m2b, axis=0, keepdims=True)

        l1 = jnp.sum((cum1 - 1.0) * m1b, axis=1, keepdims=True)
        l2 = jnp.sum((cum2 - 1.0 + count1) * m2b, axis=1, keepdims=True)
        k1 = l1 < C
        k2 = l2 < C
        k1f = k1.astype(jnp.float32)
        k2f = k2.astype(jnp.float32)
        g1 = jnp.sum(gb * m1b, axis=1, keepdims=True) * k1f
        g2 = jnp.sum(gb * m2b, axis=1, keepdims=True) * k2f
        den = jnp.clip(g1 + g2, 1e-9, None)
        g1n = g1 / den
        g2n = g2 / den

        l1i = l1.astype(jnp.int32)
        l2i = l2.astype(jnp.int32)
        p1 = ((cio == l1i) & k1).astype(jnp.float32)
        p2 = ((cio == l2i) & k2).astype(jnp.float32)
        sids = (lax.broadcasted_iota(jnp.int32, (BS, 1), 0)
                + b * BS).astype(jnp.float32)
        m1k = m1b * k1f
        m2k = m2b * k2f
        dn = (((0,), (0,)), ((), ()))
        rhs = jnp.concatenate(
            [p1 * sids, p2 * sids, p1 * g1n, p2 * g2n], axis=1)
        lhs = jnp.concatenate([m1k, m2k], axis=1)
        R = lax.dot_general(lhs, rhs, dn, precision=_HI)
        tfs_acc = tfs_acc + R[:E, :C] + R[E:, C:2 * C]
        gfs_acc = gfs_acc + R[:E, 2 * C:3 * C] + R[E:, 3 * C:]
        used = used + jnp.sum(m1k + m2k, axis=0, keepdims=True)

        raw1_l.append(e1b * C + l1i)
        raw2_l.append(e2b * C + l2i)
        k1_l.append(k1)
        k2_l.append(k2)

    umin = jnp.min(used, axis=1, keepdims=True)
    eio1 = lax.broadcasted_iota(jnp.int32, (1, E), 1).astype(jnp.float32)
    smin_e = jnp.min(jnp.where(used == umin, eio1, jnp.float32(E)),
                     axis=1, keepdims=True).astype(jnp.int32)
    sentinel = smin_e * C + (C - 1)

    raw1 = jnp.concatenate(raw1_l, axis=0)
    raw2 = jnp.concatenate(raw2_l, axis=0)
    keep1 = jnp.concatenate(k1_l, axis=0)
    keep2 = jnp.concatenate(k2_l, axis=0)
    slot1_ref[...] = jnp.where(keep1, raw1, sentinel)
    slot2_ref[...] = jnp.where(keep2, raw2, sentinel)
    tfs_ref[...] = tfs_acc.astype(jnp.int32)
    gfs_ref[...] = gfs_acc


def _gating(x2d, wg, interpret=False):
    return pl.pallas_call(
        _gating_body,
        out_shape=(
            jax.ShapeDtypeStruct((S, 1), jnp.int32),
            jax.ShapeDtypeStruct((S, 1), jnp.int32),
            jax.ShapeDtypeStruct((E, C), jnp.int32),
            jax.ShapeDtypeStruct((E, C), jnp.float32),
        ),
        interpret=interpret,
    )(x2d, wg)


NSLOT_H = NSLOT // 2


def _dispatch_body(x_hbm, tfs_hbm, buf_hbm, idx_v, r0, r1, gsem, wsem):
    wid = lax.axis_index("s") * 2 + lax.axis_index("c")
    base = wid * (NSLOT_H // NW)
    pltpu.sync_copy(tfs_hbm.at[pl.ds(base, 32)], idx_v.at[0])
    pltpu.sync_copy(tfs_hbm.at[pl.ds(base + 32, 32)], idx_v.at[1])
    g0 = pltpu.async_copy(x_hbm.at[idx_v.at[0]], r0, gsem)
    g1 = pltpu.async_copy(x_hbm.at[idx_v.at[1]], r1, gsem)
    g0.wait()
    w0 = pltpu.async_copy(r0, buf_hbm.at[pl.ds(base, 32)], wsem)
    g1.wait()
    w1 = pltpu.async_copy(r1, buf_hbm.at[pl.ds(base + 32, 32)], wsem)
    w0.wait()
    w1.wait()


def _dispatch(x2d, tfs_half):
    mesh = plsc.VectorSubcoreMesh(core_axis_name="c", subcore_axis_name="s")
    f = pl.kernel(
        _dispatch_body,
        out_type=jax.ShapeDtypeStruct((NSLOT_H, D), jnp.float32),
        mesh=mesh,
        scratch_types=[
            pltpu.VMEM((2, 32), jnp.int32),
            pltpu.VMEM((32, D), jnp.float32),
            pltpu.VMEM((32, D), jnp.float32),
            pltpu.SemaphoreType.DMA,
            pltpu.SemaphoreType.DMA,
        ],
    )
    return f(x2d, tfs_half)


FB = 2048
NF = F // FB


def _ffn_body(d_ref, w1_ref, w2_ref, gfs_ref, o_ref):
    fb = pl.program_id(1)
    d16 = d_ref[...].astype(jnp.bfloat16)
    w1b = w1_ref[0].astype(jnp.bfloat16)
    h = jnp.dot(d16, w1b, preferred_element_type=jnp.float32)
    h = jax.nn.gelu(h)
    pp = jnp.dot(h.astype(jnp.bfloat16), w2_ref[0].astype(jnp.bfloat16),
                 preferred_element_type=jnp.float32)

    @pl.when(fb == 0)
    def _():
        o_ref[...] = pp

    @pl.when(fb > 0)
    def _():
        o_ref[...] = o_ref[...] + pp

    @pl.when(fb == NF - 1)
    def _():
        o_ref[...] = o_ref[...] * gfs_ref[...]


def _ffn_half_body(d_ref, w1_ref, w2_ref, gfs_ref, eo_prev_ref, o_ref):
    _ffn_body(d_ref, w1_ref, w2_ref, gfs_ref, o_ref)


def _ffn_half(buf_half, w1, w2, gfs_col, e_off, eo_prev=None, interpret=False):
    eh = E // 2
    in_specs = [
        pl.BlockSpec((C, D), lambda e, f: (e, 0)),
        pl.BlockSpec((1, D, FB), lambda e, f: (e + e_off, 0, f)),
        pl.BlockSpec((1, FB, D), lambda e, f: (e + e_off, f, 0)),
        pl.BlockSpec((C, 1), lambda e, f: (e + e_off, 0)),
    ]
    args = [buf_half, w1, w2, gfs_col]
    body = _ffn_body
    aliases = {}
    if eo_prev is not None:
        in_specs.append(pl.BlockSpec(memory_space=pl.ANY))
        args.append(eo_prev)
        body = _ffn_half_body
        aliases = {4: 0}
    return pl.pallas_call(
        body,
        grid=(eh, NF),
        in_specs=in_specs,
        out_specs=pl.BlockSpec((C, D), lambda e, f: (e + e_off, 0)),
        out_shape=jax.ShapeDtypeStruct((NSLOT, D), jnp.float32),
        input_output_aliases=aliases,
        compiler_params=pltpu.CompilerParams(
            dimension_semantics=("parallel", "arbitrary")),
        interpret=interpret,
    )(*args)


def _add_rows(a_v, b_v):
    @plsc.parallel_loop(0, 32 * (D // 16), 1, unroll=8)
    def _(j):
        t = lax.shift_right_logical(j, 6)
        k = pl.multiple_of(
            lax.shift_left(jnp.bitwise_and(j, D // 16 - 1), 4), 16)
        a_v[t, pl.ds(k, 16)] = a_v[t, pl.ds(k, 16)] + b_v[t, pl.ds(k, 16)]


def _combine_body(eo_hbm, s1_hbm, s2_hbm, out_hbm,
                  i1_v, i2_v, a0, a1, b_v, gsem, wsem):
    wid = lax.axis_index("s") * 2 + lax.axis_index("c")
    base = wid * TOK_PER_W
    pltpu.sync_copy(s1_hbm.at[pl.ds(base, TOK_PER_W)], i1_v)
    pltpu.sync_copy(s2_hbm.at[pl.ds(base, TOK_PER_W)], i2_v)
    ga = pltpu.async_copy(eo_hbm.at[i1_v.at[pl.ds(0, 32)]], a0, gsem)
    gb = pltpu.async_copy(eo_hbm.at[i2_v.at[pl.ds(0, 32)]], b_v, gsem)
    ga.wait()
    gb.wait()
    _add_rows(a0, b_v)
    w0 = pltpu.async_copy(a0, out_hbm.at[pl.ds(base, 32)], wsem)
    ga = pltpu.async_copy(eo_hbm.at[i1_v.at[pl.ds(32, 32)]], a1, gsem)
    gb = pltpu.async_copy(eo_hbm.at[i2_v.at[pl.ds(32, 32)]], b_v, gsem)
    ga.wait()
    gb.wait()
    _add_rows(a1, b_v)
    w0.wait()
    pltpu.sync_copy(a1, out_hbm.at[pl.ds(base + 32, 32)])


def _combine(eo, s1, s2):
    mesh = plsc.VectorSubcoreMesh(core_axis_name="c", subcore_axis_name="s")
    f = pl.kernel(
        _combine_body,
        out_type=jax.ShapeDtypeStruct((S, D), jnp.float32),
        mesh=mesh,
        scratch_types=[
            pltpu.VMEM((TOK_PER_W,), jnp.int32),
            pltpu.VMEM((TOK_PER_W,), jnp.int32),
            pltpu.VMEM((32, D), jnp.float32),
            pltpu.VMEM((32, D), jnp.float32),
            pltpu.VMEM((32, D), jnp.float32),
            pltpu.SemaphoreType.DMA,
            pltpu.SemaphoreType.DMA,
        ],
    )
    return f(eo, s1, s2)


def kernel(input, wg, w1, w2):
    x2d = input.reshape(S, D)
    slot1, slot2, tfs, gfs = _gating(x2d, wg)
    tfs1 = tfs.reshape(NSLOT)
    gcol = gfs.reshape(NSLOT, 1)
    buf_a = _dispatch(x2d, tfs1[:NSLOT_H])
    buf_b = _dispatch(x2d, tfs1[NSLOT_H:])
    eo_a = _ffn_half(buf_a, w1, w2, gcol, 0)
    eo = _ffn_half(buf_b, w1, w2, gcol, E // 2, eo_a)
    out = _combine(eo, slot1.reshape(S), slot2.reshape(S))
    return out.reshape(1, S, D)


# speedup vs baseline: 1.0108x; 1.0108x over previous
import functools

import jax
import jax.numpy as jnp
from jax import lax
from jax.experimental import pallas as pl
from jax.experimental.pallas import tpu as pltpu
from jax.experimental.pallas import tpu_sc as plsc

S = 2048
E = 16
C = 256
D = 1024
F = 4096
NSLOT = E * C

NW = 32
SLOTS_PER_W = NSLOT // NW
TOK_PER_W = S // NW

_HI = jax.lax.Precision.HIGHEST


def _gating_body(x_ref, wg_ref, slot1_ref, slot2_ref, tfs_ref, gfs_ref):
    logits = jnp.dot(x_ref[...], wg_ref[...],
                     preferred_element_type=jnp.float32)
    gates = jax.nn.softmax(logits, axis=-1)

    eio = lax.broadcasted_iota(jnp.int32, (S, E), 1)
    m1 = jnp.max(gates, axis=1, keepdims=True)
    idx1 = jnp.min(jnp.where(gates == m1, eio, E), axis=1, keepdims=True)
    mask1 = (eio == idx1).astype(jnp.float32)
    logits2 = jnp.where(mask1 > 0, -jnp.inf, logits)
    m2 = jnp.max(logits2, axis=1, keepdims=True)
    idx2 = jnp.min(jnp.where(logits2 == m2, eio, E), axis=1, keepdims=True)
    mask2 = (eio == idx2).astype(jnp.float32)

    count1 = jnp.sum(mask1, axis=0, keepdims=True)

    NB = 8
    BS = S // NB
    tril = (lax.broadcasted_iota(jnp.int32, (BS, BS), 0)
            >= lax.broadcasted_iota(jnp.int32, (BS, BS), 1)
            ).astype(jnp.float32)

    carry1 = jnp.zeros((1, E), jnp.float32)
    carry2 = jnp.zeros((1, E), jnp.float32)
    tfs_acc = jnp.zeros((E, C), jnp.float32)
    gfs_acc = jnp.zeros((E, C), jnp.float32)
    used = jnp.zeros((1, E), jnp.float32)
    raw1_l, raw2_l, k1_l, k2_l = [], [], [], []

    cio = lax.broadcasted_iota(jnp.int32, (BS, C), 1)

    for b in range(NB):
        sl = slice(b * BS, (b + 1) * BS)
        m1b, m2b, gb = mask1[sl], mask2[sl], gates[sl]
        e1b, e2b = idx1[sl], idx2[sl]
        m12 = jnp.concatenate([m1b, m2b], axis=1)
        cum12 = (jnp.dot(tril, m12, precision=_HI)
                 + jnp.concatenate([carry1, carry2], axis=1))
        cum1 = cum12[:, :E]
        cum2 = cum12[:, E:]
        carry1 = carry1 + jnp.sum(m1b, axis=0, keepdims=True)
        carry2 = carry2 + jnp.sum(m2b, axis=0, keepdims=True)

        l1 = jnp.sum((cum1 - 1.0) * m1b, axis=1, keepdims=True)
        l2 = jnp.sum((cum2 - 1.0 + count1) * m2b, axis=1, keepdims=True)
        k1 = l1 < C
        k2 = l2 < C
        k1f = k1.astype(jnp.float32)
        k2f = k2.astype(jnp.float32)
        g1 = jnp.sum(gb * m1b, axis=1, keepdims=True) * k1f
        g2 = jnp.sum(gb * m2b, axis=1, keepdims=True) * k2f
        den = jnp.clip(g1 + g2, 1e-9, None)
        g1n = g1 / den
        g2n = g2 / den

        l1i = l1.astype(jnp.int32)
        l2i = l2.astype(jnp.int32)
        p1 = ((cio == l1i) & k1).astype(jnp.float32)
        p2 = ((cio == l2i) & k2).astype(jnp.float32)
        sids = (lax.broadcasted_iota(jnp.int32, (BS, 1), 0)
                + b * BS).astype(jnp.float32)
        m1k = m1b * k1f
        m2k = m2b * k2f
        dn = (((0,), (0,)), ((), ()))
        rhs = jnp.concatenate(
            [p1 * sids, p2 * sids, p1 * g1n, p2 * g2n], axis=1)
        lhs = jnp.concatenate([m1k, m2k], axis=1)
        R = lax.dot_general(lhs, rhs, dn, precision=_HI)
        tfs_acc = tfs_acc + R[:E, :C] + R[E:, C:2 * C]
        gfs_acc = gfs_acc + R[:E, 2 * C:3 * C] + R[E:, 3 * C:]
        used = used + jnp.sum(m1k + m2k, axis=0, keepdims=True)

        raw1_l.append(e1b * C + l1i)
        raw2_l.append(e2b * C + l2i)
        k1_l.append(k1)
        k2_l.append(k2)

    umin = jnp.min(used, axis=1, keepdims=True)
    eio1 = lax.broadcasted_iota(jnp.int32, (1, E), 1).astype(jnp.float32)
    smin_e = jnp.min(jnp.where(used == umin, eio1, jnp.float32(E)),
                     axis=1, keepdims=True).astype(jnp.int32)
    sentinel = smin_e * C + (C - 1)

    raw1 = jnp.concatenate(raw1_l, axis=0)
    raw2 = jnp.concatenate(raw2_l, axis=0)
    keep1 = jnp.concatenate(k1_l, axis=0)
    keep2 = jnp.concatenate(k2_l, axis=0)
    slot1_ref[...] = jnp.where(keep1, raw1, sentinel)
    slot2_ref[...] = jnp.where(keep2, raw2, sentinel)
    tfs_ref[...] = tfs_acc.astype(jnp.int32)
    gfs_ref[...] = gfs_acc


def _gating(x2d, wg, interpret=False):
    return pl.pallas_call(
        _gating_body,
        out_shape=(
            jax.ShapeDtypeStruct((S, 1), jnp.int32),
            jax.ShapeDtypeStruct((S, 1), jnp.int32),
            jax.ShapeDtypeStruct((E, C), jnp.int32),
            jax.ShapeDtypeStruct((E, C), jnp.float32),
        ),
        interpret=interpret,
    )(x2d, wg)


def _dispatch_body(x_hbm, tfs_hbm, buf_hbm, idx_v, r0, r1, gsem, wsem):
    wid = lax.axis_index("s") * 2 + lax.axis_index("c")
    base = wid * SLOTS_PER_W
    rows = (r0, r1)
    pltpu.sync_copy(tfs_hbm.at[pl.ds(base, SLOTS_PER_W)], idx_v)
    g = [None] * 4
    w = [None] * 4
    g[0] = pltpu.async_copy(x_hbm.at[idx_v.at[pl.ds(0, 32)]], rows[0], gsem)
    g[1] = pltpu.async_copy(x_hbm.at[idx_v.at[pl.ds(32, 32)]], rows[1], gsem)
    for c in range(4):
        g[c].wait()
        w[c] = pltpu.async_copy(
            rows[c % 2], buf_hbm.at[pl.ds(base + c * 32, 32)], wsem)
        if c + 2 < 4:
            w[c].wait()
            g[c + 2] = pltpu.async_copy(
                x_hbm.at[idx_v.at[pl.ds((c + 2) * 32, 32)]], rows[c % 2], gsem)
    w[2].wait()
    w[3].wait()


def _dispatch(x2d, tfs1d):
    mesh = plsc.VectorSubcoreMesh(core_axis_name="c", subcore_axis_name="s")
    f = pl.kernel(
        _dispatch_body,
        out_type=jax.ShapeDtypeStruct((NSLOT, D), jnp.float32),
        mesh=mesh,
        scratch_types=[
            pltpu.VMEM((SLOTS_PER_W,), jnp.int32),
            pltpu.VMEM((32, D), jnp.float32),
            pltpu.VMEM((32, D), jnp.float32),
            pltpu.SemaphoreType.DMA,
            pltpu.SemaphoreType.DMA,
        ],
    )
    return f(x2d, tfs1d)


FB = 2048
NF = F // FB


def _ffn_body(d_ref, w1_ref, w2_ref, gfs_ref, o_ref):
    fb = pl.program_id(1)
    d16 = d_ref[...].astype(jnp.bfloat16)
    w1b = w1_ref[0].astype(jnp.bfloat16)
    h = jnp.dot(d16, w1b, preferred_element_type=jnp.float32)
    h = jax.nn.gelu(h)
    pp = jnp.dot(h.astype(jnp.bfloat16), w2_ref[0].astype(jnp.bfloat16),
                 preferred_element_type=jnp.float32)

    @pl.when(fb == 0)
    def _():
        o_ref[...] = pp

    @pl.when(fb > 0)
    def _():
        o_ref[...] = o_ref[...] + pp

    @pl.when(fb == NF - 1)
    def _():
        o_ref[...] = o_ref[...] * gfs_ref[...]


def _ffn(buf, w1, w2, gfs_col, interpret=False):
    return pl.pallas_call(
        _ffn_body,
        grid=(E, NF),
        in_specs=[
            pl.BlockSpec((C, D), lambda e, f: (e, 0)),
            pl.BlockSpec((1, D, FB), lambda e, f: (e, 0, f)),
            pl.BlockSpec((1, FB, D), lambda e, f: (e, f, 0)),
            pl.BlockSpec((C, 1), lambda e, f: (e, 0)),
        ],
        out_specs=pl.BlockSpec((C, D), lambda e, f: (e, 0)),
        out_shape=jax.ShapeDtypeStruct((NSLOT, D), jnp.float32),
        compiler_params=pltpu.CompilerParams(
            dimension_semantics=("parallel", "arbitrary")),
        interpret=interpret,
    )(buf, w1, w2, gfs_col)


def _add_rows(a_v, b_v):
    @plsc.parallel_loop(0, 32 * (D // 16), 1, unroll=8)
    def _(j):
        t = lax.shift_right_logical(j, 6)
        k = pl.multiple_of(
            lax.shift_left(jnp.bitwise_and(j, D // 16 - 1), 4), 16)
        a_v[t, pl.ds(k, 16)] = a_v[t, pl.ds(k, 16)] + b_v[t, pl.ds(k, 16)]


def _combine_body(eo_hbm, s1_hbm, s2_hbm, out_hbm,
                  i1_v, i2_v, a0, a1, b_v, gsem, wsem):
    wid = lax.axis_index("s") * 2 + lax.axis_index("c")
    base = wid * TOK_PER_W
    pltpu.sync_copy(s1_hbm.at[pl.ds(base, TOK_PER_W)], i1_v)
    pltpu.sync_copy(s2_hbm.at[pl.ds(base, TOK_PER_W)], i2_v)
    ga = pltpu.async_copy(eo_hbm.at[i1_v.at[pl.ds(0, 32)]], a0, gsem)
    gb = pltpu.async_copy(eo_hbm.at[i2_v.at[pl.ds(0, 32)]], b_v, gsem)
    ga.wait()
    gb.wait()
    _add_rows(a0, b_v)
    w0 = pltpu.async_copy(a0, out_hbm.at[pl.ds(base, 32)], wsem)
    ga = pltpu.async_copy(eo_hbm.at[i1_v.at[pl.ds(32, 32)]], a1, gsem)
    gb = pltpu.async_copy(eo_hbm.at[i2_v.at[pl.ds(32, 32)]], b_v, gsem)
    ga.wait()
    gb.wait()
    _add_rows(a1, b_v)
    w0.wait()
    pltpu.sync_copy(a1, out_hbm.at[pl.ds(base + 32, 32)])


def _combine(eo, s1, s2):
    mesh = plsc.VectorSubcoreMesh(core_axis_name="c", subcore_axis_name="s")
    f = pl.kernel(
        _combine_body,
        out_type=jax.ShapeDtypeStruct((S, D), jnp.float32),
        mesh=mesh,
        scratch_types=[
            pltpu.VMEM((TOK_PER_W,), jnp.int32),
            pltpu.VMEM((TOK_PER_W,), jnp.int32),
            pltpu.VMEM((32, D), jnp.float32),
            pltpu.VMEM((32, D), jnp.float32),
            pltpu.VMEM((32, D), jnp.float32),
            pltpu.SemaphoreType.DMA,
            pltpu.SemaphoreType.DMA,
        ],
    )
    return f(eo, s1, s2)


def kernel(input, wg, w1, w2):
    x2d = input.reshape(S, D)
    slot1, slot2, tfs, gfs = _gating(x2d, wg)
    buf = _dispatch(x2d, tfs.reshape(NSLOT))
    eo = _ffn(buf, w1, w2, gfs.reshape(NSLOT, 1))
    out = _combine(eo, slot1.reshape(S), slot2.reshape(S))
    return out.reshape(1, S, D)


# speedup vs baseline: 1.0216x; 1.0106x over previous
import functools

import jax
import jax.numpy as jnp
from jax import lax
from jax.experimental import pallas as pl
from jax.experimental.pallas import tpu as pltpu
from jax.experimental.pallas import tpu_sc as plsc

S = 2048
E = 16
C = 256
D = 1024
F = 4096
NSLOT = E * C

NW = 32
SLOTS_PER_W = NSLOT // NW
TOK_PER_W = S // NW

_HI = jax.lax.Precision.HIGHEST


def _gating_body(x_ref, wg_ref, slot1_ref, slot2_ref, tfs_ref, gfs_ref):
    logits = jnp.dot(x_ref[...], wg_ref[...],
                     preferred_element_type=jnp.float32)
    gates = jax.nn.softmax(logits, axis=-1)

    eio = lax.broadcasted_iota(jnp.int32, (S, E), 1)
    m1 = jnp.max(gates, axis=1, keepdims=True)
    idx1 = jnp.min(jnp.where(gates == m1, eio, E), axis=1, keepdims=True)
    mask1 = (eio == idx1).astype(jnp.float32)
    logits2 = jnp.where(mask1 > 0, -jnp.inf, logits)
    m2 = jnp.max(logits2, axis=1, keepdims=True)
    idx2 = jnp.min(jnp.where(logits2 == m2, eio, E), axis=1, keepdims=True)
    mask2 = (eio == idx2).astype(jnp.float32)

    count1 = jnp.sum(mask1, axis=0, keepdims=True)

    NB = 8
    BS = S // NB
    tril = (lax.broadcasted_iota(jnp.int32, (BS, BS), 0)
            >= lax.broadcasted_iota(jnp.int32, (BS, BS), 1)
            ).astype(jnp.float32)

    carry1 = jnp.zeros((1, E), jnp.float32)
    carry2 = jnp.zeros((1, E), jnp.float32)
    tfs_acc = jnp.zeros((E, C), jnp.float32)
    gfs_acc = jnp.zeros((E, C), jnp.float32)
    used = jnp.zeros((1, E), jnp.float32)
    raw1_l, raw2_l, k1_l, k2_l = [], [], [], []

    cio = lax.broadcasted_iota(jnp.int32, (BS, C), 1)

    for b in range(NB):
        sl = slice(b * BS, (b + 1) * BS)
        m1b, m2b, gb = mask1[sl], mask2[sl], gates[sl]
        e1b, e2b = idx1[sl], idx2[sl]
        cum1 = jnp.dot(tril, m1b, precision=_HI) + carry1
        cum2 = jnp.dot(tril, m2b, precision=_HI) + carry2
        carry1 = carry1 + jnp.sum(m1b, axis=0, keepdims=True)
        carry2 = carry2 + jnp.sum(m2b, axis=0, keepdims=True)

        l1 = jnp.sum((cum1 - 1.0) * m1b, axis=1, keepdims=True)
        l2 = jnp.sum((cum2 - 1.0 + count1) * m2b, axis=1, keepdims=True)
        k1 = l1 < C
        k2 = l2 < C
        k1f = k1.astype(jnp.float32)
        k2f = k2.astype(jnp.float32)
        g1 = jnp.sum(gb * m1b, axis=1, keepdims=True) * k1f
        g2 = jnp.sum(gb * m2b, axis=1, keepdims=True) * k2f
        den = jnp.clip(g1 + g2, 1e-9, None)
        g1n = g1 / den
        g2n = g2 / den

        l1i = l1.astype(jnp.int32)
        l2i = l2.astype(jnp.int32)
        p1 = ((cio == l1i) & k1).astype(jnp.float32)
        p2 = ((cio == l2i) & k2).astype(jnp.float32)
        sids = (lax.broadcasted_iota(jnp.int32, (BS, 1), 0)
                + b * BS).astype(jnp.float32)
        m1k = m1b * k1f
        m2k = m2b * k2f
        dn = (((0,), (0,)), ((), ()))
        tfs_acc = tfs_acc + lax.dot_general(m1k, p1 * sids, dn, precision=_HI)
        tfs_acc = tfs_acc + lax.dot_general(m2k, p2 * sids, dn, precision=_HI)
        gfs_acc = gfs_acc + lax.dot_general(m1k, p1 * g1n, dn, precision=_HI)
        gfs_acc = gfs_acc + lax.dot_general(m2k, p2 * g2n, dn, precision=_HI)
        used = used + jnp.sum(m1k + m2k, axis=0, keepdims=True)

        raw1_l.append(e1b * C + l1i)
        raw2_l.append(e2b * C + l2i)
        k1_l.append(k1)
        k2_l.append(k2)

    umin = jnp.min(used, axis=1, keepdims=True)
    eio1 = lax.broadcasted_iota(jnp.int32, (1, E), 1).astype(jnp.float32)
    smin_e = jnp.min(jnp.where(used == umin, eio1, jnp.float32(E)),
                     axis=1, keepdims=True).astype(jnp.int32)
    sentinel = smin_e * C + (C - 1)

    raw1 = jnp.concatenate(raw1_l, axis=0)
    raw2 = jnp.concatenate(raw2_l, axis=0)
    keep1 = jnp.concatenate(k1_l, axis=0)
    keep2 = jnp.concatenate(k2_l, axis=0)
    slot1_ref[...] = jnp.where(keep1, raw1, sentinel)
    slot2_ref[...] = jnp.where(keep2, raw2, sentinel)
    tfs_ref[...] = tfs_acc.astype(jnp.int32)
    gfs_ref[...] = gfs_acc


def _gating(x2d, wg, interpret=False):
    return pl.pallas_call(
        _gating_body,
        out_shape=(
            jax.ShapeDtypeStruct((S, 1), jnp.int32),
            jax.ShapeDtypeStruct((S, 1), jnp.int32),
            jax.ShapeDtypeStruct((E, C), jnp.int32),
            jax.ShapeDtypeStruct((E, C), jnp.float32),
        ),
        interpret=interpret,
    )(x2d, wg)


def _dispatch_body(x_hbm, tfs_hbm, buf_hbm, idx_v, r0, r1, gsem, wsem):
    wid = lax.axis_index("s") * 2 + lax.axis_index("c")
    base = wid * SLOTS_PER_W
    rows = (r0, r1)
    pltpu.sync_copy(tfs_hbm.at[pl.ds(base, SLOTS_PER_W)], idx_v)
    g = [None] * 4
    w = [None] * 4
    g[0] = pltpu.async_copy(x_hbm.at[idx_v.at[pl.ds(0, 32)]], rows[0], gsem)
    g[1] = pltpu.async_copy(x_hbm.at[idx_v.at[pl.ds(32, 32)]], rows[1], gsem)
    for c in range(4):
        g[c].wait()
        w[c] = pltpu.async_copy(
            rows[c % 2], buf_hbm.at[pl.ds(base + c * 32, 32)], wsem)
        if c + 2 < 4:
            w[c].wait()
            g[c + 2] = pltpu.async_copy(
                x_hbm.at[idx_v.at[pl.ds((c + 2) * 32, 32)]], rows[c % 2], gsem)
    w[2].wait()
    w[3].wait()


def _dispatch(x2d, tfs1d):
    mesh = plsc.VectorSubcoreMesh(core_axis_name="c", subcore_axis_name="s")
    f = pl.kernel(
        _dispatch_body,
        out_type=jax.ShapeDtypeStruct((NSLOT, D), jnp.float32),
        mesh=mesh,
        scratch_types=[
            pltpu.VMEM((SLOTS_PER_W,), jnp.int32),
            pltpu.VMEM((32, D), jnp.float32),
            pltpu.VMEM((32, D), jnp.float32),
            pltpu.SemaphoreType.DMA,
            pltpu.SemaphoreType.DMA,
        ],
    )
    return f(x2d, tfs1d)


FB = 2048
NF = F // FB


def _ffn_body(d_ref, w1_ref, w2_ref, gfs_ref, o_ref):
    fb = pl.program_id(1)
    d16 = d_ref[...].astype(jnp.bfloat16)
    w1b = w1_ref[0].astype(jnp.bfloat16)
    h = jnp.dot(d16, w1b, preferred_element_type=jnp.float32)
    h = jax.nn.gelu(h)
    pp = jnp.dot(h.astype(jnp.bfloat16), w2_ref[0].astype(jnp.bfloat16),
                 preferred_element_type=jnp.float32)

    @pl.when(fb == 0)
    def _():
        o_ref[...] = pp

    @pl.when(fb > 0)
    def _():
        o_ref[...] = o_ref[...] + pp

    @pl.when(fb == NF - 1)
    def _():
        o_ref[...] = o_ref[...] * gfs_ref[...]


def _ffn(buf, w1, w2, gfs_col, interpret=False):
    return pl.pallas_call(
        _ffn_body,
        grid=(E, NF),
        in_specs=[
            pl.BlockSpec((C, D), lambda e, f: (e, 0)),
            pl.BlockSpec((1, D, FB), lambda e, f: (e, 0, f)),
            pl.BlockSpec((1, FB, D), lambda e, f: (e, f, 0)),
            pl.BlockSpec((C, 1), lambda e, f: (e, 0)),
        ],
        out_specs=pl.BlockSpec((C, D), lambda e, f: (e, 0)),
        out_shape=jax.ShapeDtypeStruct((NSLOT, D), jnp.float32),
        compiler_params=pltpu.CompilerParams(
            dimension_semantics=("parallel", "arbitrary")),
        interpret=interpret,
    )(buf, w1, w2, gfs_col)


def _add_rows(a_v, b_v):
    @plsc.parallel_loop(0, 32 * (D // 16), 1, unroll=8)
    def _(j):
        t = lax.shift_right_logical(j, 6)
        k = pl.multiple_of(
            lax.shift_left(jnp.bitwise_and(j, D // 16 - 1), 4), 16)
        a_v[t, pl.ds(k, 16)] = a_v[t, pl.ds(k, 16)] + b_v[t, pl.ds(k, 16)]


def _combine_body(eo_hbm, s1_hbm, s2_hbm, out_hbm,
                  i1_v, i2_v, a0, a1, b_v, gsem, wsem):
    wid = lax.axis_index("s") * 2 + lax.axis_index("c")
    base = wid * TOK_PER_W
    pltpu.sync_copy(s1_hbm.at[pl.ds(base, TOK_PER_W)], i1_v)
    pltpu.sync_copy(s2_hbm.at[pl.ds(base, TOK_PER_W)], i2_v)
    ga = pltpu.async_copy(eo_hbm.at[i1_v.at[pl.ds(0, 32)]], a0, gsem)
    gb = pltpu.async_copy(eo_hbm.at[i2_v.at[pl.ds(0, 32)]], b_v, gsem)
    ga.wait()
    gb.wait()
    _add_rows(a0, b_v)
    w0 = pltpu.async_copy(a0, out_hbm.at[pl.ds(base, 32)], wsem)
    ga = pltpu.async_copy(eo_hbm.at[i1_v.at[pl.ds(32, 32)]], a1, gsem)
    gb = pltpu.async_copy(eo_hbm.at[i2_v.at[pl.ds(32, 32)]], b_v, gsem)
    ga.wait()
    gb.wait()
    _add_rows(a1, b_v)
    w0.wait()
    pltpu.sync_copy(a1, out_hbm.at[pl.ds(base + 32, 32)])


def _combine(eo, s1, s2):
    mesh = plsc.VectorSubcoreMesh(core_axis_name="c", subcore_axis_name="s")
    f = pl.kernel(
        _combine_body,
        out_type=jax.ShapeDtypeStruct((S, D), jnp.float32),
        mesh=mesh,
        scratch_types=[
            pltpu.VMEM((TOK_PER_W,), jnp.int32),
            pltpu.VMEM((TOK_PER_W,), jnp.int32),
            pltpu.VMEM((32, D), jnp.float32),
            pltpu.VMEM((32, D), jnp.float32),
            pltpu.VMEM((32, D), jnp.float32),
            pltpu.SemaphoreType.DMA,
            pltpu.SemaphoreType.DMA,
        ],
    )
    return f(eo, s1, s2)


def kernel(input, wg, w1, w2):
    x2d = input.reshape(S, D)
    slot1, slot2, tfs, gfs = _gating(x2d, wg)
    buf = _dispatch(x2d, tfs.reshape(NSLOT))
    eo = _ffn(buf, w1, w2, gfs.reshape(NSLOT, 1))
    out = _combine(eo, slot1.reshape(S), slot2.reshape(S))
    return out.reshape(1, S, D)


# speedup vs baseline: 1.0222x; 1.0006x over previous
import functools

import jax
import jax.numpy as jnp
from jax import lax
from jax.experimental import pallas as pl
from jax.experimental.pallas import tpu as pltpu
from jax.experimental.pallas import tpu_sc as plsc

S = 2048
E = 16
C = 256
D = 1024
F = 4096
NSLOT = E * C

NW = 32
SLOTS_PER_W = NSLOT // NW
TOK_PER_W = S // NW

_HI = jax.lax.Precision.HIGHEST


def _gating_body(x_ref, wg_ref, slot1_ref, slot2_ref, tfs_ref, gfs_ref):
    logits = jnp.dot(x_ref[...], wg_ref[...],
                     preferred_element_type=jnp.float32)
    gates = jax.nn.softmax(logits, axis=-1)

    eio = lax.broadcasted_iota(jnp.int32, (S, E), 1)
    m1 = jnp.max(gates, axis=1, keepdims=True)
    idx1 = jnp.min(jnp.where(gates == m1, eio, E), axis=1, keepdims=True)
    mask1 = (eio == idx1).astype(jnp.float32)
    logits2 = jnp.where(mask1 > 0, -jnp.inf, logits)
    m2 = jnp.max(logits2, axis=1, keepdims=True)
    idx2 = jnp.min(jnp.where(logits2 == m2, eio, E), axis=1, keepdims=True)
    mask2 = (eio == idx2).astype(jnp.float32)

    count1 = jnp.sum(mask1, axis=0, keepdims=True)

    NB = 8
    BS = S // NB
    tril = (lax.broadcasted_iota(jnp.int32, (BS, BS), 0)
            >= lax.broadcasted_iota(jnp.int32, (BS, BS), 1)
            ).astype(jnp.float32)

    carry1 = jnp.zeros((1, E), jnp.float32)
    carry2 = jnp.zeros((1, E), jnp.float32)
    tfs_acc = jnp.zeros((E, C), jnp.float32)
    gfs_acc = jnp.zeros((E, C), jnp.float32)
    used = jnp.zeros((1, E), jnp.float32)
    raw1_l, raw2_l, k1_l, k2_l = [], [], [], []

    cio = lax.broadcasted_iota(jnp.int32, (BS, C), 1)

    for b in range(NB):
        sl = slice(b * BS, (b + 1) * BS)
        m1b, m2b, gb = mask1[sl], mask2[sl], gates[sl]
        e1b, e2b = idx1[sl], idx2[sl]
        cum1 = jnp.dot(tril, m1b, precision=_HI) + carry1
        cum2 = jnp.dot(tril, m2b, precision=_HI) + carry2
        carry1 = carry1 + jnp.sum(m1b, axis=0, keepdims=True)
        carry2 = carry2 + jnp.sum(m2b, axis=0, keepdims=True)

        l1 = jnp.sum((cum1 - 1.0) * m1b, axis=1, keepdims=True)
        l2 = jnp.sum((cum2 - 1.0 + count1) * m2b, axis=1, keepdims=True)
        k1 = l1 < C
        k2 = l2 < C
        k1f = k1.astype(jnp.float32)
        k2f = k2.astype(jnp.float32)
        g1 = jnp.sum(gb * m1b, axis=1, keepdims=True) * k1f
        g2 = jnp.sum(gb * m2b, axis=1, keepdims=True) * k2f
        den = jnp.clip(g1 + g2, 1e-9, None)
        g1n = g1 / den
        g2n = g2 / den

        l1i = l1.astype(jnp.int32)
        l2i = l2.astype(jnp.int32)
        p1 = ((cio == l1i) & k1).astype(jnp.float32)
        p2 = ((cio == l2i) & k2).astype(jnp.float32)
        sids = (lax.broadcasted_iota(jnp.int32, (BS, 1), 0)
                + b * BS).astype(jnp.float32)
        m1k = m1b * k1f
        m2k = m2b * k2f
        dn = (((0,), (0,)), ((), ()))
        tfs_acc = tfs_acc + lax.dot_general(m1k, p1 * sids, dn, precision=_HI)
        tfs_acc = tfs_acc + lax.dot_general(m2k, p2 * sids, dn, precision=_HI)
        gfs_acc = gfs_acc + lax.dot_general(m1k, p1 * g1n, dn, precision=_HI)
        gfs_acc = gfs_acc + lax.dot_general(m2k, p2 * g2n, dn, precision=_HI)
        used = used + jnp.sum(m1k + m2k, axis=0, keepdims=True)

        raw1_l.append(e1b * C + l1i)
        raw2_l.append(e2b * C + l2i)
        k1_l.append(k1)
        k2_l.append(k2)

    umin = jnp.min(used, axis=1, keepdims=True)
    eio1 = lax.broadcasted_iota(jnp.int32, (1, E), 1).astype(jnp.float32)
    smin_e = jnp.min(jnp.where(used == umin, eio1, jnp.float32(E)),
                     axis=1, keepdims=True).astype(jnp.int32)
    sentinel = smin_e * C + (C - 1)

    raw1 = jnp.concatenate(raw1_l, axis=0)
    raw2 = jnp.concatenate(raw2_l, axis=0)
    keep1 = jnp.concatenate(k1_l, axis=0)
    keep2 = jnp.concatenate(k2_l, axis=0)
    slot1_ref[...] = jnp.where(keep1, raw1, sentinel)
    slot2_ref[...] = jnp.where(keep2, raw2, sentinel)
    tfs_ref[...] = tfs_acc.astype(jnp.int32)
    gfs_ref[...] = gfs_acc


def _gating(x2d, wg):
    return pl.pallas_call(
        _gating_body,
        out_shape=(
            jax.ShapeDtypeStruct((S, 1), jnp.int32),
            jax.ShapeDtypeStruct((S, 1), jnp.int32),
            jax.ShapeDtypeStruct((E, C), jnp.int32),
            jax.ShapeDtypeStruct((E, C), jnp.float32),
        ),
    )(x2d, wg)


def _dispatch_body(x_hbm, tfs_hbm, buf_hbm, idx_v, r0, r1, gsem, wsem):
    wid = lax.axis_index("s") * 2 + lax.axis_index("c")
    base = wid * SLOTS_PER_W
    rows = (r0, r1)
    pltpu.sync_copy(tfs_hbm.at[pl.ds(base, SLOTS_PER_W)], idx_v)
    g = [None] * 4
    w = [None] * 4
    g[0] = pltpu.async_copy(x_hbm.at[idx_v.at[pl.ds(0, 32)]], rows[0], gsem)
    g[1] = pltpu.async_copy(x_hbm.at[idx_v.at[pl.ds(32, 32)]], rows[1], gsem)
    for c in range(4):
        g[c].wait()
        w[c] = pltpu.async_copy(
            rows[c % 2], buf_hbm.at[pl.ds(base + c * 32, 32)], wsem)
        if c + 2 < 4:
            w[c].wait()
            g[c + 2] = pltpu.async_copy(
                x_hbm.at[idx_v.at[pl.ds((c + 2) * 32, 32)]], rows[c % 2], gsem)
    w[2].wait()
    w[3].wait()


def _dispatch(x2d, tfs1d):
    mesh = plsc.VectorSubcoreMesh(core_axis_name="c", subcore_axis_name="s")
    f = pl.kernel(
        _dispatch_body,
        out_type=jax.ShapeDtypeStruct((NSLOT, D), jnp.float32),
        mesh=mesh,
        scratch_types=[
            pltpu.VMEM((SLOTS_PER_W,), jnp.int32),
            pltpu.VMEM((32, D), jnp.float32),
            pltpu.VMEM((32, D), jnp.float32),
            pltpu.SemaphoreType.DMA,
            pltpu.SemaphoreType.DMA,
        ],
    )
    return f(x2d, tfs1d)


FB = 2048
NF = F // FB


def _ffn_body(d_ref, w1_ref, w2_ref, gfs_ref, o_ref):
    fb = pl.program_id(1)
    d16 = d_ref[...].astype(jnp.bfloat16)
    w1b = w1_ref[0].astype(jnp.bfloat16)
    h = jnp.dot(d16, w1b, preferred_element_type=jnp.float32)
    h = jax.nn.gelu(h)
    pp = jnp.dot(h.astype(jnp.bfloat16), w2_ref[0].astype(jnp.bfloat16),
                 preferred_element_type=jnp.float32)

    @pl.when(fb == 0)
    def _():
        o_ref[...] = pp

    @pl.when(fb > 0)
    def _():
        o_ref[...] = o_ref[...] + pp

    @pl.when(fb == NF - 1)
    def _():
        o_ref[...] = o_ref[...] * gfs_ref[...]


def _ffn(buf, w1, w2, gfs_col):
    return pl.pallas_call(
        _ffn_body,
        grid=(E, NF),
        in_specs=[
            pl.BlockSpec((C, D), lambda e, f: (e, 0)),
            pl.BlockSpec((1, D, FB), lambda e, f: (e, 0, f)),
            pl.BlockSpec((1, FB, D), lambda e, f: (e, f, 0)),
            pl.BlockSpec((C, 1), lambda e, f: (e, 0)),
        ],
        out_specs=pl.BlockSpec((C, D), lambda e, f: (e, 0)),
        out_shape=jax.ShapeDtypeStruct((NSLOT, D), jnp.float32),
        compiler_params=pltpu.CompilerParams(
            dimension_semantics=("parallel", "arbitrary")),
    )(buf, w1, w2, gfs_col)


def _add_rows(a_v, b_v):
    @plsc.parallel_loop(0, 32 * (D // 16), 1, unroll=8)
    def _(j):
        t = lax.shift_right_logical(j, 6)
        k = pl.multiple_of(
            lax.shift_left(jnp.bitwise_and(j, D // 16 - 1), 4), 16)
        a_v[t, pl.ds(k, 16)] = a_v[t, pl.ds(k, 16)] + b_v[t, pl.ds(k, 16)]


def _combine_body(eo_hbm, s1_hbm, s2_hbm, out_hbm,
                  i1_v, i2_v, a0, a1, b_v, gsem, wsem):
    wid = lax.axis_index("s") * 2 + lax.axis_index("c")
    base = wid * TOK_PER_W
    pltpu.sync_copy(s1_hbm.at[pl.ds(base, TOK_PER_W)], i1_v)
    pltpu.sync_copy(s2_hbm.at[pl.ds(base, TOK_PER_W)], i2_v)
    ga = pltpu.async_copy(eo_hbm.at[i1_v.at[pl.ds(0, 32)]], a0, gsem)
    gb = pltpu.async_copy(eo_hbm.at[i2_v.at[pl.ds(0, 32)]], b_v, gsem)
    ga.wait()
    gb.wait()
    _add_rows(a0, b_v)
    w0 = pltpu.async_copy(a0, out_hbm.at[pl.ds(base, 32)], wsem)
    ga = pltpu.async_copy(eo_hbm.at[i1_v.at[pl.ds(32, 32)]], a1, gsem)
    gb = pltpu.async_copy(eo_hbm.at[i2_v.at[pl.ds(32, 32)]], b_v, gsem)
    ga.wait()
    gb.wait()
    _add_rows(a1, b_v)
    w0.wait()
    pltpu.sync_copy(a1, out_hbm.at[pl.ds(base + 32, 32)])


def _combine(eo, s1, s2):
    mesh = plsc.VectorSubcoreMesh(core_axis_name="c", subcore_axis_name="s")
    f = pl.kernel(
        _combine_body,
        out_type=jax.ShapeDtypeStruct((S, D), jnp.float32),
        mesh=mesh,
        scratch_types=[
            pltpu.VMEM((TOK_PER_W,), jnp.int32),
            pltpu.VMEM((TOK_PER_W,), jnp.int32),
            pltpu.VMEM((32, D), jnp.float32),
            pltpu.VMEM((32, D), jnp.float32),
            pltpu.VMEM((32, D), jnp.float32),
            pltpu.SemaphoreType.DMA,
            pltpu.SemaphoreType.DMA,
        ],
    )
    return f(eo, s1, s2)


def kernel(input, wg, w1, w2):
    x2d = input.reshape(S, D)
    slot1, slot2, tfs, gfs = _gating(x2d, wg)
    buf = _dispatch(x2d, tfs.reshape(NSLOT))
    eo = _ffn(buf, w1, w2, gfs.reshape(NSLOT, 1))
    out = _combine(eo, slot1.reshape(S), slot2.reshape(S))
    return out.reshape(1, S, D)
